# baseline (speedup 1.0000x reference)
import functools

import jax
import jax.numpy as jnp
from jax import lax
from jax.experimental import pallas as pl
from jax.experimental.pallas import tpu as pltpu
from jax.experimental.pallas import tpu_sc as plsc


def _coef_body(wt_ref, ct_ref):
    w = wt_ref[...]
    m = jnp.max(w, axis=0, keepdims=True)
    e = jnp.exp(w - m)
    s = jnp.sum(e, axis=0, keepdims=True)
    c0 = e[8] + e[9] + e[10] + e[11] + e[12] + e[13] + e[14] + e[15]
    ca = e[2] + e[3] + e[6] + e[7] - e[8] - e[9] - e[12] - e[13]
    cb = e[4] + e[5] + e[6] + e[7] - e[8] - e[9] - e[10] - e[11]
    cab = (e[1] - e[2] - e[4] - 2.0 * e[6] - e[7]
           + e[8] + 2.0 * e[9] + e[11] + e[13] - e[14])
    ct_ref[...] = jnp.stack([c0, ca, cb, cab], axis=0) / s


def _coef_table(wt):
    n = wt.shape[1]
    return pl.pallas_call(
        _coef_body,
        out_shape=jax.ShapeDtypeStruct((4, n), jnp.float32),
    )(wt)


R = 8
SB = 512
NG = SB // 16


def _logic_body(x_hbm, ct_hbm, idx_hbm, out_hbm,
                idx0_v, idx1_v, c0_v, ca_v, cb_v, cab_v,
                xbuf, obuf, sem_x, sem_o,
                *, batch, in_dim, out_dim, nc):
    rpw = batch // (nc * 16)
    nchunk = rpw // R
    nsb = out_dim // SB
    wid = lax.axis_index("s") * nc + lax.axis_index("c")
    base_row = wid * rpw

    pltpu.sync_copy(idx_hbm.at[0], idx0_v)
    pltpu.sync_copy(idx_hbm.at[1], idx1_v)
    pltpu.sync_copy(ct_hbm.at[0], c0_v)
    pltpu.sync_copy(ct_hbm.at[1], ca_v)
    pltpu.sync_copy(ct_hbm.at[2], cb_v)
    pltpu.sync_copy(ct_hbm.at[3], cab_v)

    pltpu.async_copy(x_hbm.at[pl.ds(base_row, R)], xbuf.at[pl.ds(0, R)],
                     sem_x.at[0])

    def chunk_body(ch, _):
        cur = lax.rem(ch, 2)
        rowbase = base_row + ch * R
        pltpu.make_async_copy(x_hbm.at[pl.ds(rowbase, R)],
                              xbuf.at[pl.ds(cur * R, R)],
                              sem_x.at[cur]).wait()

        @pl.when(ch + 1 < nchunk)
        def _prefetch():
            nxt = lax.rem(ch + 1, 2)
            pltpu.async_copy(x_hbm.at[pl.ds(rowbase + R, R)],
                             xbuf.at[pl.ds(nxt * R, R)],
                             sem_x.at[nxt])

        rowv = [jnp.full((16,), cur * R + r, dtype=jnp.int32) for r in range(R)]

        def sb_body(sb, _):
            t = ch * nsb + sb
            ob = lax.rem(sb, 2)

            @pl.when(t >= 2)
            def _drain():
                tp = t - 2
                chp = tp // nsb
                sbp = lax.rem(tp, nsb)
                pltpu.make_async_copy(
                    obuf.at[ob],
                    out_hbm.at[pl.ds(base_row + chp * R, R),
                               pl.ds(sbp * SB, SB)],
                    sem_o.at[ob]).wait()

            def g_body(g, _):
                jb = sb * SB + g * 16
                i0 = idx0_v[pl.ds(jb, 16)]
                i1 = idx1_v[pl.ds(jb, 16)]
                k0 = c0_v[pl.ds(jb, 16)]
                ka = ca_v[pl.ds(jb, 16)]
                kb = cb_v[pl.ds(jb, 16)]
                kab = cab_v[pl.ds(jb, 16)]
                for r in range(R):
                    a = plsc.load_gather(xbuf, [rowv[r], i0])
                    b = plsc.load_gather(xbuf, [rowv[r], i1])
                    val = k0 + a * ka + b * kb + (a * b) * kab
                    obuf[ob, r, pl.ds(g * 16, 16)] = val
                return _

            lax.fori_loop(0, NG, g_body, None)
            pltpu.async_copy(obuf.at[ob],
                             out_hbm.at[pl.ds(rowbase, R), pl.ds(sb * SB, SB)],
                             sem_o.at[ob])
            return _

        lax.fori_loop(0, nsb, sb_body, None)
        return _

    lax.fori_loop(0, nchunk, chunk_body, None)

    lastrow = base_row + (nchunk - 1) * R
    for sb in (nsb - 2, nsb - 1):
        pltpu.make_async_copy(obuf.at[sb % 2],
                              out_hbm.at[pl.ds(lastrow, R),
                                         pl.ds(sb * SB, SB)],
                              sem_o.at[sb % 2]).wait()


def _logic_dense_sc(x, ct, idx):
    batch, in_dim = x.shape
    out_dim = idx.shape[1]
    mesh = plsc.VectorSubcoreMesh(core_axis_name="c", subcore_axis_name="s")
    nc = mesh.num_cores

    body = functools.partial(_logic_body, batch=batch, in_dim=in_dim,
                             out_dim=out_dim, nc=nc)
    f = pl.kernel(
        body,
        out_type=jax.ShapeDtypeStruct((batch, out_dim), jnp.float32),
        mesh=mesh,
        compiler_params=pltpu.CompilerParams(use_tc_tiling_on_sc=False,
                                             needs_layout_passes=False),
        scratch_types=[
            pltpu.VMEM((out_dim,), jnp.int32),
            pltpu.VMEM((out_dim,), jnp.int32),
            pltpu.VMEM((out_dim,), jnp.float32),
            pltpu.VMEM((out_dim,), jnp.float32),
            pltpu.VMEM((out_dim,), jnp.float32),
            pltpu.VMEM((out_dim,), jnp.float32),
            pltpu.VMEM((2 * R, in_dim), jnp.float32),
            pltpu.VMEM((2, R, SB), jnp.float32),
            pltpu.SemaphoreType.DMA((2,)),
            pltpu.SemaphoreType.DMA((2,)),
        ],
    )
    return f(x, ct, idx)


def kernel(x, weight, indices):
    ct = _coef_table(weight.T.astype(jnp.float32))
    idx = indices.astype(jnp.int32)
    return _logic_dense_sc(x, ct, idx)


# speedup vs baseline: 4.0418x; 4.0418x over previous
import functools

import jax
import jax.numpy as jnp
from jax import lax
from jax.experimental import pallas as pl
from jax.experimental.pallas import tpu as pltpu
from jax.experimental.pallas import tpu_sc as plsc


def _coef_body(wt_ref, idx_ref, c0_ref, ca_ref, cb_ref, cab_ref, idxp_ref):
    w = wt_ref[...]
    m = jnp.max(w, axis=0, keepdims=True)
    e = jnp.exp(w - m)
    rs = 1.0 / jnp.sum(e, axis=0)
    c0_ref[...] = (e[8] + e[9] + e[10] + e[11]
                   + e[12] + e[13] + e[14] + e[15]) * rs
    ca_ref[...] = (e[2] + e[3] + e[6] + e[7]
                   - e[8] - e[9] - e[12] - e[13]) * rs
    cb_ref[...] = (e[4] + e[5] + e[6] + e[7]
                   - e[8] - e[9] - e[10] - e[11]) * rs
    cab_ref[...] = (e[1] - e[2] - e[4] - 2.0 * e[6] - e[7]
                    + e[8] + 2.0 * e[9] + e[11] + e[13] - e[14]) * rs
    idxp_ref[...] = jnp.bitwise_or(jnp.left_shift(idx_ref[1], 12), idx_ref[0])


def _coef_table(wt, idx):
    n = wt.shape[1]
    return pl.pallas_call(
        _coef_body,
        out_shape=[jax.ShapeDtypeStruct((n,), jnp.float32)] * 4
        + [jax.ShapeDtypeStruct((n,), jnp.int32)],
    )(wt, idx)


R = 8
SB = 1024
NG = SB // 16


def _logic_body(x_hbm, c0_hbm, ca_hbm, cb_hbm, cab_hbm, idxp_hbm,
                out_hbm,
                idxp_v, c0_v, ca_v, cb_v, cab_v,
                xbuf, obuf, sem_x, sem_o,
                *, batch, in_dim, out_dim, nc):
    rpw = batch // (nc * 16)
    nchunk = rpw // R
    nsb = out_dim // SB
    wid = lax.axis_index("s") * nc + lax.axis_index("c")
    base_row = wid * rpw

    pltpu.sync_copy(idxp_hbm, idxp_v)
    pltpu.sync_copy(c0_hbm, c0_v)
    pltpu.sync_copy(ca_hbm, ca_v)
    pltpu.sync_copy(cb_hbm, cb_v)
    pltpu.sync_copy(cab_hbm, cab_v)

    pltpu.async_copy(x_hbm.at[pl.ds(base_row, R)], xbuf.at[pl.ds(0, R)],
                     sem_x.at[0])

    def chunk_body(ch, _):
        cur = lax.rem(ch, 2)
        rowbase = base_row + ch * R
        pltpu.make_async_copy(x_hbm.at[pl.ds(rowbase, R)],
                              xbuf.at[pl.ds(cur * R, R)],
                              sem_x.at[cur]).wait()

        @pl.when(ch + 1 < nchunk)
        def _prefetch():
            nxt = lax.rem(ch + 1, 2)
            pltpu.async_copy(x_hbm.at[pl.ds(rowbase + R, R)],
                             xbuf.at[pl.ds(nxt * R, R)],
                             sem_x.at[nxt])

        rowv = [jnp.full((16,), cur * R + r, dtype=jnp.int32) for r in range(R)]

        def sb_body(sb, _):
            t = ch * nsb + sb
            ob = lax.rem(sb, 2)

            @pl.when(t >= 2)
            def _drain():
                tp = t - 2
                chp = tp // nsb
                sbp = lax.rem(tp, nsb)
                pltpu.make_async_copy(
                    obuf.at[ob],
                    out_hbm.at[pl.ds(base_row + chp * R, R),
                               pl.ds(sbp * SB, SB)],
                    sem_o.at[ob]).wait()

            @plsc.parallel_loop(0, NG, unroll=2)
            def g_body(g):
                jb = sb * SB + g * 16
                ip = idxp_v[pl.ds(jb, 16)]
                i0 = jnp.bitwise_and(ip, 4095)
                i1 = jnp.right_shift(ip, 12)
                k0 = c0_v[pl.ds(jb, 16)]
                ka = ca_v[pl.ds(jb, 16)]
                kb = cb_v[pl.ds(jb, 16)]
                kab = cab_v[pl.ds(jb, 16)]
                ab = [(plsc.load_gather(xbuf, [rowv[r], i0]),
                       plsc.load_gather(xbuf, [rowv[r], i1]))
                      for r in range(R)]
                for r in range(R):
                    a, b = ab[r]
                    val = (k0 + a * ka) + b * (kb + a * kab)
                    obuf[ob, r, pl.ds(g * 16, 16)] = val
            pltpu.async_copy(obuf.at[ob],
                             out_hbm.at[pl.ds(rowbase, R), pl.ds(sb * SB, SB)],
                             sem_o.at[ob])
            return _

        lax.fori_loop(0, nsb, sb_body, None)
        return _

    lax.fori_loop(0, nchunk, chunk_body, None)

    lastrow = base_row + (nchunk - 1) * R
    for sb in (nsb - 2, nsb - 1):
        pltpu.make_async_copy(obuf.at[sb % 2],
                              out_hbm.at[pl.ds(lastrow, R),
                                         pl.ds(sb * SB, SB)],
                              sem_o.at[sb % 2]).wait()


def _logic_dense_sc(x, c0, ca, cb, cab, idxp):
    batch, in_dim = x.shape
    out_dim = idxp.shape[0]
    mesh = plsc.VectorSubcoreMesh(core_axis_name="c", subcore_axis_name="s")
    nc = mesh.num_cores

    body = functools.partial(_logic_body, batch=batch, in_dim=in_dim,
                             out_dim=out_dim, nc=nc)
    f = pl.kernel(
        body,
        out_type=jax.ShapeDtypeStruct((batch, out_dim), jnp.float32),
        mesh=mesh,
        compiler_params=pltpu.CompilerParams(use_tc_tiling_on_sc=True,
                                             needs_layout_passes=False),
        scratch_types=[
            pltpu.VMEM((out_dim,), jnp.int32),
            pltpu.VMEM((out_dim,), jnp.float32),
            pltpu.VMEM((out_dim,), jnp.float32),
            pltpu.VMEM((out_dim,), jnp.float32),
            pltpu.VMEM((out_dim,), jnp.float32),
            pltpu.VMEM((2 * R, in_dim), jnp.float32),
            pltpu.VMEM((2, R, SB), jnp.float32),
            pltpu.SemaphoreType.DMA((2,)),
            pltpu.SemaphoreType.DMA((2,)),
        ],
    )
    return f(x, c0, ca, cb, cab, idxp)


def kernel(x, weight, indices):
    c0, ca, cb, cab, idxp = _coef_table(weight.T.astype(jnp.float32),
                                        indices.astype(jnp.int32))
    return _logic_dense_sc(x, c0, ca, cb, cab, idxp)


# speedup vs baseline: 4.1024x; 1.0150x over previous
import functools

import jax
import jax.numpy as jnp
from jax import lax
from jax.experimental import pallas as pl
from jax.experimental.pallas import tpu as pltpu
from jax.experimental.pallas import tpu_sc as plsc


def _coef_body(wt_ref, idx_ref, c0_ref, ca_ref, cb_ref, cab_ref, idxp_ref):
    w = wt_ref[...]
    m = jnp.max(w, axis=0, keepdims=True)
    e = jnp.exp(w - m)
    rs = 1.0 / jnp.sum(e, axis=0)
    c0_ref[...] = (e[8] + e[9] + e[10] + e[11]
                   + e[12] + e[13] + e[14] + e[15]) * rs
    ca_ref[...] = (e[2] + e[3] + e[6] + e[7]
                   - e[8] - e[9] - e[12] - e[13]) * rs
    cb_ref[...] = (e[4] + e[5] + e[6] + e[7]
                   - e[8] - e[9] - e[10] - e[11]) * rs
    cab_ref[...] = (e[1] - e[2] - e[4] - 2.0 * e[6] - e[7]
                    + e[8] + 2.0 * e[9] + e[11] + e[13] - e[14]) * rs
    idxp_ref[...] = jnp.bitwise_or(jnp.left_shift(idx_ref[1], 12), idx_ref[0])


def _coef_table(wt, idx):
    n = wt.shape[1]
    return pl.pallas_call(
        _coef_body,
        out_shape=[jax.ShapeDtypeStruct((n,), jnp.float32)] * 4
        + [jax.ShapeDtypeStruct((n,), jnp.int32)],
    )(wt, idx)


R = 8
SB = 1024
NG = SB // 16


def _logic_body(x_hbm, c0_hbm, ca_hbm, cb_hbm, cab_hbm, idxp_hbm,
                out_hbm,
                idxp_v, c0_v, ca_v, cb_v, cab_v,
                xbuf, obuf, sem_x, sem_o,
                *, batch, in_dim, out_dim, nc):
    rpw = batch // (nc * 16)
    nchunk = rpw // R
    nsb = out_dim // SB
    wid = lax.axis_index("s") * nc + lax.axis_index("c")
    base_row = wid * rpw

    pltpu.async_copy(x_hbm.at[pl.ds(base_row, R)], xbuf.at[pl.ds(0, R)],
                     sem_x.at[0])
    tbl = [(idxp_hbm, idxp_v), (c0_hbm, c0_v), (ca_hbm, ca_v),
           (cb_hbm, cb_v), (cab_hbm, cab_v)]
    for src, dst in tbl:
        pltpu.async_copy(src, dst, sem_o.at[0])
    for src, dst in tbl:
        pltpu.make_async_copy(src, dst, sem_o.at[0]).wait()

    def chunk_body(ch, _):
        cur = lax.rem(ch, 2)
        rowbase = base_row + ch * R
        pltpu.make_async_copy(x_hbm.at[pl.ds(rowbase, R)],
                              xbuf.at[pl.ds(cur * R, R)],
                              sem_x.at[cur]).wait()

        @pl.when(ch + 1 < nchunk)
        def _prefetch():
            nxt = lax.rem(ch + 1, 2)
            pltpu.async_copy(x_hbm.at[pl.ds(rowbase + R, R)],
                             xbuf.at[pl.ds(nxt * R, R)],
                             sem_x.at[nxt])

        rowv = [jnp.full((16,), cur * R + r, dtype=jnp.int32) for r in range(R)]

        def sb_body(sb, _):
            t = ch * nsb + sb
            ob = lax.rem(sb, 2)

            @pl.when(t >= 2)
            def _drain():
                tp = t - 2
                chp = tp // nsb
                sbp = lax.rem(tp, nsb)
                pltpu.make_async_copy(
                    obuf.at[ob],
                    out_hbm.at[pl.ds(base_row + chp * R, R),
                               pl.ds(sbp * SB, SB)],
                    sem_o.at[ob]).wait()

            @plsc.parallel_loop(0, NG, unroll=2)
            def g_body(g):
                jb = sb * SB + g * 16
                ip = idxp_v[pl.ds(jb, 16)]
                i0 = jnp.bitwise_and(ip, 4095)
                i1 = jnp.right_shift(ip, 12)
                k0 = c0_v[pl.ds(jb, 16)]
                ka = ca_v[pl.ds(jb, 16)]
                kb = cb_v[pl.ds(jb, 16)]
                kab = cab_v[pl.ds(jb, 16)]
                ab = [(plsc.load_gather(xbuf, [rowv[r], i0]),
                       plsc.load_gather(xbuf, [rowv[r], i1]))
                      for r in range(R)]
                for r in range(R):
                    a, b = ab[r]
                    val = (k0 + a * ka) + b * (kb + a * kab)
                    obuf[ob, r, pl.ds(g * 16, 16)] = val
            pltpu.async_copy(obuf.at[ob],
                             out_hbm.at[pl.ds(rowbase, R), pl.ds(sb * SB, SB)],
                             sem_o.at[ob])
            return _

        lax.fori_loop(0, nsb, sb_body, None)
        return _

    lax.fori_loop(0, nchunk, chunk_body, None)

    lastrow = base_row + (nchunk - 1) * R
    for sb in (nsb - 2, nsb - 1):
        pltpu.make_async_copy(obuf.at[sb % 2],
                              out_hbm.at[pl.ds(lastrow, R),
                                         pl.ds(sb * SB, SB)],
                              sem_o.at[sb % 2]).wait()


def _logic_dense_sc(x, c0, ca, cb, cab, idxp):
    batch, in_dim = x.shape
    out_dim = idxp.shape[0]
    mesh = plsc.VectorSubcoreMesh(core_axis_name="c", subcore_axis_name="s")
    nc = mesh.num_cores

    body = functools.partial(_logic_body, batch=batch, in_dim=in_dim,
                             out_dim=out_dim, nc=nc)
    f = pl.kernel(
        body,
        out_type=jax.ShapeDtypeStruct((batch, out_dim), jnp.float32),
        mesh=mesh,
        compiler_params=pltpu.CompilerParams(use_tc_tiling_on_sc=True,
                                             needs_layout_passes=False),
        scratch_types=[
            pltpu.VMEM((out_dim,), jnp.int32),
            pltpu.VMEM((out_dim,), jnp.float32),
            pltpu.VMEM((out_dim,), jnp.float32),
            pltpu.VMEM((out_dim,), jnp.float32),
            pltpu.VMEM((out_dim,), jnp.float32),
            pltpu.VMEM((2 * R, in_dim), jnp.float32),
            pltpu.VMEM((2, R, SB), jnp.float32),
            pltpu.SemaphoreType.DMA((2,)),
            pltpu.SemaphoreType.DMA((2,)),
        ],
    )
    return f(x, c0, ca, cb, cab, idxp)


def kernel(x, weight, indices):
    c0, ca, cb, cab, idxp = _coef_table(weight.T.astype(jnp.float32),
                                        indices.astype(jnp.int32))
    return _logic_dense_sc(x, c0, ca, cb, cab, idxp)
